# baseline (speedup 1.0000x reference)
import jax
from jax.experimental import pallas as pl
from jax.experimental.pallas import tpu as pltpu

_BLKC = 1280


def _body(xt_blk, o_blk):
    o_blk[...] = xt_blk[3:, :].T


def _kern(xt_hbm, o_hbm):
    f, n = xt_hbm.shape
    pltpu.emit_pipeline(
        _body,
        grid=(pl.cdiv(n, _BLKC),),
        in_specs=[pl.BlockSpec((f, _BLKC), lambda i: (0, i))],
        out_specs=[pl.BlockSpec((_BLKC, f - 3), lambda i: (i, 0))],
    )(xt_hbm, o_hbm)


def kernel(x, W, b):
    n, f = x.shape
    fo = f - 3
    xt = x.T
    return pl.pallas_call(
        _kern,
        in_specs=[pl.BlockSpec(memory_space=pltpu.MemorySpace.HBM)],
        out_specs=pl.BlockSpec(memory_space=pltpu.MemorySpace.HBM),
        out_shape=jax.ShapeDtypeStruct((n, fo), x.dtype),
    )(xt)


# speedup vs baseline: 1.6946x; 1.6946x over previous
import jax
from jax.experimental import pallas as pl

_BLKC = 5120


def _slice_transpose_kernel(xt_ref, o_ref):
    o_ref[...] = xt_ref[3:, :].T


def kernel(x, W, b):
    n, f = x.shape
    fo = f - 3
    xt = x.T
    return pl.pallas_call(
        _slice_transpose_kernel,
        grid=(pl.cdiv(n, _BLKC),),
        in_specs=[pl.BlockSpec((f, _BLKC), lambda i: (0, i))],
        out_specs=pl.BlockSpec((_BLKC, fo), lambda i: (i, 0)),
        out_shape=jax.ShapeDtypeStruct((n, fo), x.dtype),
    )(xt)
